# Initial kernel scaffold; baseline (speedup 1.0000x reference)
import functools

import jax
import jax.numpy as jnp
from jax import lax
from jax.experimental import pallas as pl
from jax.experimental.pallas import tpu as pltpu
from jax.experimental.pallas import tpu_sc as plsc

NC = 2
NS = 16
L = 16
NW = NC * NS

D = 128
NN = 32
G = 4
ROWS = G * NN


def _gather_mean(x, nodes3, neigh3, b_pad):
    b_per_w = b_pad // NW
    n_groups = b_per_w // G
    mesh = plsc.VectorSubcoreMesh(core_axis_name="c", subcore_axis_name="s")

    @functools.partial(
        pl.kernel,
        mesh=mesh,
        out_type=jax.ShapeDtypeStruct((b_pad, 2 * D), jnp.float32),
        scratch_types=[
            pltpu.VMEM((n_groups, ROWS), jnp.int32),
            pltpu.VMEM((b_per_w,), jnp.int32),
            pltpu.VMEM((b_per_w, D), jnp.float32),
            pltpu.VMEM((ROWS, D), jnp.float32),
            pltpu.VMEM((G, D), jnp.float32),
            pltpu.SemaphoreType.DMA,
        ],
    )
    def k(x_hbm, nodes_hbm, neigh_hbm, h_hbm, nidx_v, sidx_v, sbuf, nbuf, hbuf, sem):
        wid = lax.axis_index("s") * NC + lax.axis_index("c")
        base_row = wid * b_per_w
        pltpu.sync_copy(neigh_hbm.at[wid], nidx_v)
        pltpu.sync_copy(nodes_hbm.at[wid], sidx_v)
        for lo in range(0, b_per_w, 128):
            sz = min(128, b_per_w - lo)
            pltpu.async_copy(
                x_hbm.at[sidx_v.at[pl.ds(lo, sz)]], sbuf.at[pl.ds(lo, sz)], sem
            ).wait()
        pltpu.sync_copy(sbuf, h_hbm.at[pl.ds(base_row, b_per_w), pl.ds(0, D)])

        inv = jnp.float32(1.0 / NN)

        def group(g, carry):
            pltpu.async_copy(x_hbm.at[nidx_v.at[g]], nbuf, sem).wait()
            for si in range(G):
                def body(n, accs):
                    row = si * NN + n
                    return tuple(
                        accs[ci] + nbuf[row, pl.ds(ci * L, L)] for ci in range(D // L)
                    )
                accs = lax.fori_loop(
                    0, NN, body, tuple(jnp.zeros((L,), jnp.float32) for _ in range(D // L))
                )
                for ci in range(D // L):
                    hbuf[si, pl.ds(ci * L, L)] = accs[ci] * inv
            pltpu.sync_copy(
                hbuf, h_hbm.at[pl.ds(base_row + g * G, G), pl.ds(D, D)]
            )
            return carry

        lax.fori_loop(0, n_groups, group, 0)

    return k(x, nodes3, neigh3)


def _mm_body(h_ref, w_ref, b_ref, o_ref):
    o_ref[...] = (
        lax.dot_general(
            h_ref[...],
            w_ref[...],
            (((1,), (0,)), ((), ())),
            preferred_element_type=jnp.float32,
            precision=lax.Precision.HIGHEST,
        )
        + b_ref[...]
    )


def _linear(h, W, b, n_out):
    b_pad = h.shape[0]
    blk = 1024
    grid = b_pad // blk
    return pl.pallas_call(
        _mm_body,
        grid=(grid,),
        in_specs=[
            pl.BlockSpec((blk, 2 * D), lambda i: (i, 0)),
            pl.BlockSpec((2 * D, D), lambda i: (0, 0)),
            pl.BlockSpec((1, D), lambda i: (0, 0)),
        ],
        out_specs=pl.BlockSpec((blk, D), lambda i: (i, 0)),
        out_shape=jax.ShapeDtypeStruct((n_out, D), jnp.float32),
    )(h, W, b.reshape(1, D))


def kernel(x, nodes, neigh_idx, W, b):
    B, n_neigh = neigh_idx.shape
    assert n_neigh == NN and x.shape[1] == D
    chunk = NW * G
    b_pad = ((B + chunk - 1) // chunk) * chunk
    pad = b_pad - B
    nodes_p = jnp.concatenate([nodes, jnp.zeros((pad,), jnp.int32)])
    neigh_p = jnp.concatenate([neigh_idx, jnp.zeros((pad, NN), jnp.int32)], axis=0)
    b_per_w = b_pad // NW
    nodes3 = nodes_p.reshape(NW, b_per_w)
    neigh3 = neigh_p.reshape(NW, b_per_w // G, G * NN)
    h = _gather_mean(x, nodes3, neigh3, b_pad)
    return _linear(h, W, b, B)


# speedup vs baseline: 1.0716x; 1.0716x over previous
import functools

import jax
import jax.numpy as jnp
from jax import lax
from jax.experimental import pallas as pl
from jax.experimental.pallas import tpu as pltpu
from jax.experimental.pallas import tpu_sc as plsc

NC = 2
NS = 16
L = 16
NW = NC * NS

D = 128
NN = 32
G = 4
ROWS = G * NN


def _gather_mean(x, nodes3, neigh3, b_pad):
    b_per_w = b_pad // NW
    n_groups = b_per_w // G
    mesh = plsc.VectorSubcoreMesh(core_axis_name="c", subcore_axis_name="s")

    @functools.partial(
        pl.kernel,
        mesh=mesh,
        out_type=jax.ShapeDtypeStruct((b_pad, 2 * D), jnp.float32),
        scratch_types=[
            pltpu.VMEM((n_groups, ROWS), jnp.int32),
            pltpu.VMEM((b_per_w,), jnp.int32),
            pltpu.VMEM((b_per_w, D), jnp.float32),
            pltpu.VMEM((ROWS, D), jnp.float32),
            pltpu.VMEM((G, D), jnp.float32),
            pltpu.SemaphoreType.DMA,
        ],
    )
    def k(x_hbm, nodes_hbm, neigh_hbm, h_hbm, nidx_v, sidx_v, sbuf, nbuf, hbuf, sem):
        wid = lax.axis_index("s") * NC + lax.axis_index("c")
        base_row = wid * b_per_w
        pltpu.sync_copy(neigh_hbm.at[wid], nidx_v)
        pltpu.sync_copy(nodes_hbm.at[wid], sidx_v)
        for lo in range(0, b_per_w, 128):
            sz = min(128, b_per_w - lo)
            pltpu.async_copy(
                x_hbm.at[sidx_v.at[pl.ds(lo, sz)]], sbuf.at[pl.ds(lo, sz)], sem
            ).wait()
        pltpu.sync_copy(sbuf, h_hbm.at[pl.ds(base_row, b_per_w), pl.ds(0, D)])

        inv = jnp.float32(1.0 / NN)

        def group(g, carry):
            pltpu.async_copy(x_hbm.at[nidx_v.at[g]], nbuf, sem).wait()
            for si in range(G):
                def body(n, accs):
                    row = si * NN + n
                    return tuple(
                        accs[ci] + nbuf[row, pl.ds(ci * L, L)] for ci in range(D // L)
                    )
                accs = lax.fori_loop(
                    0, NN, body, tuple(jnp.zeros((L,), jnp.float32) for _ in range(D // L))
                )
                for ci in range(D // L):
                    hbuf[si, pl.ds(ci * L, L)] = accs[ci] * inv
            pltpu.sync_copy(
                hbuf, h_hbm.at[pl.ds(base_row + g * G, G), pl.ds(D, D)]
            )
            return carry

        lax.fori_loop(0, n_groups, group, 0)

    return k(x, nodes3, neigh3)


def _mm_body(h_ref, w_ref, b_ref, o_ref):
    o_ref[...] = (
        lax.dot_general(
            h_ref[...],
            w_ref[...],
            (((1,), (0,)), ((), ())),
            preferred_element_type=jnp.float32,
            precision=lax.Precision.HIGHEST,
        )
        + b_ref[...]
    )


def _linear(h, W, b, n_out):
    b_pad = h.shape[0]
    blk = 1024
    grid = b_pad // blk
    return pl.pallas_call(
        _mm_body,
        grid=(grid,),
        in_specs=[
            pl.BlockSpec((blk, 2 * D), lambda i: (i, 0)),
            pl.BlockSpec((2 * D, D), lambda i: (0, 0)),
            pl.BlockSpec((1, D), lambda i: (0, 0)),
        ],
        out_specs=pl.BlockSpec((blk, D), lambda i: (i, 0)),
        out_shape=jax.ShapeDtypeStruct((n_out, D), jnp.float32),
    )(h, W, b.reshape(1, D))


def kernel(x, nodes, neigh_idx, W, b):
    B, n_neigh = neigh_idx.shape
    assert n_neigh == NN and x.shape[1] == D
    chunk = 1024
    b_pad = ((B + chunk - 1) // chunk) * chunk
    pad = b_pad - B
    nodes_p = jnp.concatenate([nodes, jnp.zeros((pad,), jnp.int32)])
    neigh_p = jnp.concatenate([neigh_idx, jnp.zeros((pad, NN), jnp.int32)], axis=0)
    b_per_w = b_pad // NW
    nodes3 = nodes_p.reshape(NW, b_per_w)
    neigh3 = neigh_p.reshape(NW, b_per_w // G, G * NN)
    h = _gather_mean(x, nodes3, neigh3, b_pad)
    return _linear(h, W, b, B)
